# baseline (speedup 1.0000x reference)
import jax
import jax.numpy as jnp
from jax import lax
from jax.experimental import pallas as pl
from jax.experimental.pallas import tpu as pltpu

_STATE = 128
_HID = 512
_ABS = 2048
_NAG = 32
_APA = 2

_NS = 4
_APS = 1
_HAG = _NAG // _NS
_GRID = _HAG // _APS


def _fused_kernel(*refs):
    state_ref, w1_ref, b1_ref = refs[0], refs[1], refs[2]
    w2 = refs[3:3 + _NS]
    b2 = refs[3 + _NS:3 + 2 * _NS]
    g1 = refs[3 + 2 * _NS:3 + 3 * _NS]
    g2 = refs[3 + 3 * _NS:3 + 4 * _NS]
    dmr = refs[3 + 4 * _NS:3 + 5 * _NS]
    outs = refs[3 + 5 * _NS:3 + 6 * _NS]
    x = jnp.maximum(
        jnp.dot(state_ref[...], w1_ref[...],
                preferred_element_type=jnp.float32) + b1_ref[...], 0.0)
    for s in range(_NS):
        y2 = jnp.dot(x, w2[s][...], preferred_element_type=jnp.float32)
        for a in range(_APS):
            y = y2[:, a * _ABS:(a + 1) * _ABS]
            y = y + b2[s][a] + g1[s][a] + g2[s][a]
            m = jnp.max(y, axis=1, keepdims=True)
            lane = lax.broadcasted_iota(jnp.int32, (1, _ABS), 1)
            idx = jnp.min(jnp.where(y == m, lane, _ABS), axis=1,
                          keepdims=True)
            dm = dmr[s][a]
            lane2 = lax.broadcasted_iota(jnp.int32, (_APA, _ABS), 1)
            vals = jnp.sum(jnp.where(lane2 == idx, dm, 0), axis=1,
                           keepdims=True)
            outs[s][a] = jnp.broadcast_to(vals, (_APA, 128))


def _off(s):
    return lambda i, s=s: (i + s * _GRID, 0, 0)


def kernel(state, W1, b1, W2, b2, g1, g2, decode_map):
    state2 = state.reshape(1, _STATE)
    b12 = b1.reshape(1, _HID)
    b2r = b2.reshape(_NAG, 1, _ABS)
    g1r = g1.reshape(_NAG, 1, _ABS)
    g2r = g2.reshape(_NAG, 1, _ABS)
    dm = decode_map.transpose(0, 2, 1)

    w2_specs = [pl.BlockSpec((_HID, _APS * _ABS),
                             (lambda i, s=s: (0, i + s * _GRID)))
                for s in range(_NS)]
    sm_specs = [pl.BlockSpec((_APS, 1, _ABS), _off(s)) for s in range(_NS)]
    dm_specs = [pl.BlockSpec((_APS, _APA, _ABS), _off(s)) for s in range(_NS)]
    out_specs = [pl.BlockSpec((_APS, _APA, 128), lambda i: (i, 0, 0))
                 for _ in range(_NS)]
    outs = pl.pallas_call(
        _fused_kernel,
        grid=(_GRID,),
        in_specs=(
            [pl.BlockSpec((1, _STATE), lambda i: (0, 0)),
             pl.BlockSpec((_STATE, _HID), lambda i: (0, 0)),
             pl.BlockSpec((1, _HID), lambda i: (0, 0))]
            + w2_specs + sm_specs + sm_specs + sm_specs + dm_specs
        ),
        out_specs=out_specs,
        out_shape=[jax.ShapeDtypeStruct((_HAG, _APA, 128), jnp.int32)
                   for _ in range(_NS)],
        compiler_params=pltpu.CompilerParams(
            dimension_semantics=("arbitrary",)),
    )(state2, W1, b12,
      *([W2] * _NS), *([b2r] * _NS), *([g1r] * _NS), *([g2r] * _NS),
      *([dm] * _NS))
    out = jnp.concatenate(outs, axis=0)
    return out[:, :, 0].reshape(-1)


# speedup vs baseline: 1.0040x; 1.0040x over previous
import jax
import jax.numpy as jnp
from jax import lax
from jax.experimental import pallas as pl
from jax.experimental.pallas import tpu as pltpu

_STATE = 128
_HID = 512
_ABS = 2048
_NAG = 32
_APA = 2

_NS = 2
_APS = 2
_HAG = _NAG // _NS
_GRID = _HAG // _APS


def _fused_kernel(*refs):
    state_ref, w1_ref, b1_ref = refs[0], refs[1], refs[2]
    w2 = refs[3:3 + _NS]
    b2 = refs[3 + _NS:3 + 2 * _NS]
    g1 = refs[3 + 2 * _NS:3 + 3 * _NS]
    g2 = refs[3 + 3 * _NS:3 + 4 * _NS]
    dmr = refs[3 + 4 * _NS:3 + 5 * _NS]
    outs = refs[3 + 5 * _NS:3 + 6 * _NS]
    x = jnp.maximum(
        jnp.dot(state_ref[...], w1_ref[...],
                preferred_element_type=jnp.float32) + b1_ref[...], 0.0)
    for s in range(_NS):
        y2 = jnp.dot(x, w2[s][...], preferred_element_type=jnp.float32)
        for a in range(_APS):
            y = y2[:, a * _ABS:(a + 1) * _ABS]
            y = y + b2[s][a] + g1[s][a] + g2[s][a]
            m = jnp.max(y, axis=1, keepdims=True)
            lane = lax.broadcasted_iota(jnp.int32, (1, _ABS), 1)
            idx = jnp.min(jnp.where(y == m, lane, _ABS), axis=1,
                          keepdims=True)
            dm = dmr[s][a]
            lane2 = lax.broadcasted_iota(jnp.int32, (_APA, _ABS), 1)
            vals = jnp.sum(jnp.where(lane2 == idx, dm, 0), axis=1,
                           keepdims=True)
            outs[s][a] = jnp.broadcast_to(vals, (_APA, 128))


def _off(s):
    return lambda i, s=s: (i + s * _GRID, 0, 0)


def kernel(state, W1, b1, W2, b2, g1, g2, decode_map):
    state2 = state.reshape(1, _STATE)
    b12 = b1.reshape(1, _HID)
    b2r = b2.reshape(_NAG, 1, _ABS)
    g1r = g1.reshape(_NAG, 1, _ABS)
    g2r = g2.reshape(_NAG, 1, _ABS)
    dm = decode_map.transpose(0, 2, 1)

    w2_specs = [pl.BlockSpec((_HID, _APS * _ABS),
                             (lambda i, s=s: (0, i + s * _GRID)))
                for s in range(_NS)]
    sm_specs = [pl.BlockSpec((_APS, 1, _ABS), _off(s)) for s in range(_NS)]
    dm_specs = [pl.BlockSpec((_APS, _APA, _ABS), _off(s)) for s in range(_NS)]
    out_specs = [pl.BlockSpec((_APS, _APA, 128), lambda i: (i, 0, 0))
                 for _ in range(_NS)]
    outs = pl.pallas_call(
        _fused_kernel,
        grid=(_GRID,),
        in_specs=(
            [pl.BlockSpec((1, _STATE), lambda i: (0, 0)),
             pl.BlockSpec((_STATE, _HID), lambda i: (0, 0)),
             pl.BlockSpec((1, _HID), lambda i: (0, 0))]
            + w2_specs + sm_specs + sm_specs + sm_specs + dm_specs
        ),
        out_specs=out_specs,
        out_shape=[jax.ShapeDtypeStruct((_HAG, _APA, 128), jnp.int32)
                   for _ in range(_NS)],
        compiler_params=pltpu.CompilerParams(
            dimension_semantics=("arbitrary",)),
    )(state2, W1, b12,
      *([W2] * _NS), *([b2r] * _NS), *([g1r] * _NS), *([g2r] * _NS),
      *([dm] * _NS))
    out = jnp.concatenate(outs, axis=0)
    return out[:, :, 0].reshape(-1)
